# baseline (speedup 1.0000x reference)
import functools

import jax
import jax.numpy as jnp
from jax import lax
from jax.experimental import pallas as pl
from jax.experimental.pallas import tpu as pltpu
from jax.experimental.pallas import tpu_sc as plsc

NC = 2
NS = 16
NW = NC * NS
C = 128
BLK = 8


def _sc_body(ctx, x_hbm, src_hbm, dst_hbm, sums_hbm, cnts_hbm,
             src_blk, dst_blk, rows0, rows1, ones_v,
             acc_sh, cnt_sh, sem_g0, sem_g1):
    n_pad, rpt, n_blocks = ctx
    cid = lax.axis_index("c")
    sid = lax.axis_index("s")
    wid = sid * NC + cid

    zeros16 = jnp.zeros((16,), jnp.float32)
    ones16 = jnp.ones((16,), jnp.float32)
    rows = (rows0, rows1)
    sems = (sem_g0, sem_g1)

    @pl.loop(0, C * 8)
    def _(t):
        rows0[t // 8, pl.ds(16 * (t % 8), 16)] = zeros16

    @pl.loop(0, C)
    def _(i):
        ones_v[i, :] = zeros16

    row0 = sid * rpt

    @pl.loop(0, rpt // C)
    def _(j):
        pltpu.sync_copy(rows0, acc_sh.at[pl.ds(row0 + j * C, C)])
        pltpu.sync_copy(ones_v, cnt_sh.at[pl.ds(row0 + j * C, C)])

    @pl.loop(0, C)
    def _(i):
        ones_v[i, :] = ones16

    plsc.subcore_barrier()

    def load_idx_block(blk):
        pltpu.sync_copy(src_hbm.at[wid, pl.ds(blk * BLK, BLK)], src_blk)
        pltpu.sync_copy(dst_hbm.at[wid, pl.ds(blk * BLK, BLK)], dst_blk)

    def gather_issue(j, b):
        pltpu.async_copy(x_hbm.at[src_blk.at[j]], rows[b], sems[b])

    def gather_wait(j, b):
        pltpu.make_async_copy(x_hbm.at[src_blk.at[j]], rows[b], sems[b]).wait()

    load_idx_block(0)
    gather_issue(0, 0)

    @pl.loop(0, n_blocks)
    def _(blk):
        for j in range(BLK):
            b = j % 2
            gather_wait(j, b)
            if j + 1 < BLK:
                gather_issue(j + 1, 1 - b)
            pltpu.sync_copy(rows[b], acc_sh.at[dst_blk.at[j]], add=True)
            pltpu.sync_copy(ones_v, cnt_sh.at[dst_blk.at[j]], add=True)

        @pl.when(blk < n_blocks - 1)
        def _():
            load_idx_block(blk + 1)
            gather_issue(0, 0)

    plsc.subcore_barrier()

    pltpu.sync_copy(acc_sh.at[pl.ds(row0, rpt)],
                    sums_hbm.at[cid, pl.ds(row0, rpt)])
    pltpu.sync_copy(cnt_sh.at[pl.ds(row0, rpt)],
                    cnts_hbm.at[cid, pl.ds(row0, rpt)])


def _segment_sums(x, src3, dst3, n_pad):
    n_blocks = src3.shape[1] // BLK
    rpt = n_pad // NS
    d = x.shape[1]
    mesh = plsc.VectorSubcoreMesh(core_axis_name="c", subcore_axis_name="s")
    body = functools.partial(_sc_body, (n_pad, rpt, n_blocks))
    return pl.kernel(
        body,
        out_type=(
            jax.ShapeDtypeStruct((NC, n_pad, d), jnp.float32),
            jax.ShapeDtypeStruct((NC, n_pad, 16), jnp.float32),
        ),
        mesh=mesh,
        compiler_params=pltpu.CompilerParams(use_tc_tiling_on_sc=False),
        scratch_types=[
            pltpu.VMEM((BLK, C), jnp.int32),
            pltpu.VMEM((BLK, C), jnp.int32),
            pltpu.VMEM((C, d), jnp.float32),
            pltpu.VMEM((C, d), jnp.float32),
            pltpu.VMEM((C, 16), jnp.float32),
            pltpu.VMEM_SHARED((n_pad, d), jnp.float32),
            pltpu.VMEM_SHARED((n_pad, 16), jnp.float32),
            pltpu.SemaphoreType.DMA,
            pltpu.SemaphoreType.DMA,
        ],
    )(x, src3, dst3)


def _tc_body(s_ref, c_ref, w_ref, b_ref, o_ref):
    s = s_ref[0] + s_ref[1]
    c = c_ref[0, :, 0:1] + c_ref[1, :, 0:1]
    mean = s / jnp.maximum(c, 1.0)
    h = lax.dot_general(mean, w_ref[...], (((1,), (1,)), ((), ())),
                        preferred_element_type=jnp.float32)
    out = h + b_ref[...] * jnp.minimum(c, 1.0)
    o_ref[...] = jnp.maximum(out, 0.0)


def _finish(sums, cnts, W, b, n_pad, rows_blk):
    d_in = W.shape[1]
    d_out = W.shape[0]
    grid = (n_pad // rows_blk,)
    return pl.pallas_call(
        _tc_body,
        grid=grid,
        in_specs=[
            pl.BlockSpec((NC, rows_blk, d_in), lambda i: (0, i, 0)),
            pl.BlockSpec((NC, rows_blk, 16), lambda i: (0, i, 0)),
            pl.BlockSpec((d_out, d_in), lambda i: (0, 0)),
            pl.BlockSpec((1, d_out), lambda i: (0, 0)),
        ],
        out_specs=pl.BlockSpec((rows_blk, d_out), lambda i: (i, 0)),
        out_shape=jax.ShapeDtypeStruct((n_pad, d_out), jnp.float32),
    )(sums, cnts, W, b.reshape(1, d_out))


def kernel(x, edge_index, W, b):
    n = x.shape[0]
    e = edge_index.shape[1]

    n_pad = ((n + 1) + NS * C - 1) // (NS * C) * (NS * C)
    egrp = NW * BLK * C
    e_pad = (e + egrp - 1) // egrp * egrp

    src = edge_index[0].astype(jnp.int32)
    dst = edge_index[1].astype(jnp.int32)
    pad = e_pad - e
    if pad:
        src = jnp.concatenate([src, jnp.zeros((pad,), jnp.int32)])
        dst = jnp.concatenate([dst, jnp.full((pad,), n, jnp.int32)])
    src3 = src.reshape(NW, e_pad // (NW * C), C)
    dst3 = dst.reshape(NW, e_pad // (NW * C), C)

    sums, cnts = _segment_sums(x, src3, dst3, n_pad)
    out = _finish(sums, cnts, W, b, n_pad, rows_blk=1024)
    return out[:n]
